# Initial kernel scaffold; baseline (speedup 1.0000x reference)
import jax
import jax.numpy as jnp
from jax.experimental import pallas as pl


def kernel(node_feat, edge_index, edge_vec, W1, W2, W3, Wl0, Wl1):
    raise NotImplementedError("write your pallas kernel here")


# speedup vs baseline: 1.4023x; 1.4023x over previous
import functools

import numpy as np
import jax
import jax.numpy as jnp
from jax import lax
from jax.experimental import pallas as pl
from jax.experimental.pallas import tpu as pltpu
from jax.experimental.pallas import tpu_sc as plsc

MUL = 16
F = 4 * MUL
SQRT2 = 2.0 ** 0.5
SQRT3 = 3.0 ** 0.5
SQRT5 = 5.0 ** 0.5

NC, NS = 2, 16
NW = NC * NS
K = 128
HALF = F // 2

NB_NODE = 2000
EB_EDGE = 4000


def _permute_body(x_ref, p_ref, o_ref):
    o_ref[...] = jnp.dot(x_ref[...], p_ref[...],
                         preferred_element_type=jnp.float32)


def _final_body(a_ref, wb_ref, nf_ref, o_ref):
    wb = wb_ref[...]
    lin = jnp.dot(a_ref[0], wb[:HALF, :], preferred_element_type=jnp.float32)
    lin += jnp.dot(a_ref[1], wb[HALF:, :], preferred_element_type=jnp.float32)
    o_ref[...] = lin + nf_ref[...]


def _msg_body(ev_ref, x_ref, w1_ref, w2_ref, w3_ref, o_ref):
    ev = ev_ref[...]
    ex, ey, ez = ev[:, 0:1], ev[:, 1:2], ev[:, 2:3]
    r = jnp.sqrt(ex * ex + ey * ey + ez * ez + 1e-12)
    inv = 1.0 / r
    nx, ny, nz = ex * inv, ey * inv, ez * inv
    h = jax.nn.silu(r * w1_ref[0:1, :])
    h = jax.nn.silu(jnp.dot(h, w2_ref[...],
                            preferred_element_type=jnp.float32) * 0.125)
    w = jnp.dot(h, w3_ref[...], preferred_element_type=jnp.float32) * 0.125
    w000, w011, w101, w110, w121 = (w[:, 0:16], w[:, 16:32], w[:, 32:48],
                                    w[:, 48:64], w[:, 64:80])
    x = x_ref[...]
    s = x[:, 0:16]
    vx, vy, vz = x[:, 16:32], x[:, 32:48], x[:, 48:64]
    sh1x, sh1y, sh1z = SQRT3 * nx, SQRT3 * ny, SQRT3 * nz
    dotv = sh1x * vx + sh1y * vy + sh1z * vz
    out0 = (w000 * s + w110 * dotv * (1.0 / SQRT3)) * (1.0 / SQRT2)
    s1 = SQRT5 * SQRT3 * nx * ny
    s2 = SQRT5 * SQRT3 * ny * nz
    s3 = SQRT5 * 0.5 * (3.0 * nz * nz - 1.0)
    s4 = SQRT5 * SQRT3 * nx * nz
    s5 = SQRT5 * 0.5 * SQRT3 * (nx * nx - ny * ny)
    a = s3 * (1.0 / SQRT3)
    tvx = (s5 - a) * vx + s1 * vy + s4 * vz
    tvy = s1 * vx + (-a - s5) * vy + s2 * vz
    tvz = s4 * vx + s2 * vy + 2.0 * a * vz
    ws = w011 * s
    c3 = 1.0 / SQRT3
    c5 = 1.0 / SQRT5
    out1x = (ws * sh1x + w101 * vx + w121 * tvx * c5) * c3
    out1y = (ws * sh1y + w101 * vy + w121 * tvy * c5) * c3
    out1z = (ws * sh1z + w101 * vz + w121 * tvz * c5) * c3
    o_ref[...] = jnp.concatenate([out0, out1x, out1y, out1z], axis=1)


def _perm_matrix():
    p = np.zeros((F, F), np.float32)
    p[np.arange(MUL), np.arange(MUL)] = 1.0
    for c in range(3):
        for u in range(MUL):
            p[MUL + 3 * u + c, MUL + MUL * c + u] = 1.0
    return p


def _folded_linear(wl0, wl1):
    wb = jnp.zeros((F, F), jnp.float32)
    wb = wb.at[:MUL, :MUL].set(wl0 * 0.25)
    cc, uu, vv = np.meshgrid(np.arange(3), np.arange(MUL), np.arange(MUL),
                             indexing="ij")
    rows = MUL + MUL * cc + uu
    cols = MUL + 3 * vv + cc
    vals = jnp.broadcast_to(wl1 * 0.25, (3, MUL, MUL))
    return wb.at[rows, cols].set(vals)


def kernel(node_feat, edge_index, edge_vec, W1, W2, W3, Wl0, Wl1):
    n_nodes = node_feat.shape[0]
    n_edges = edge_vec.shape[0]
    assert n_edges % K == 0
    nch = n_edges // K
    per_w = -(-nch // NW)
    per_t = -(-nch // NS)
    n_pad = -(-n_nodes // (NS * 8)) * NS * 8
    rows_t = n_pad // NS

    src_r = edge_index[0].reshape(nch, K)
    dst_r = edge_index[1].reshape(nch, K)

    mesh = plsc.VectorSubcoreMesh(core_axis_name="c", subcore_axis_name="s",
                                  num_cores=NC, num_subcores=NS)

    pmat = jnp.asarray(_perm_matrix())
    perm = pl.pallas_call(
        _permute_body,
        grid=(n_nodes // NB_NODE,),
        in_specs=[pl.BlockSpec((NB_NODE, F), lambda i: (i, 0)),
                  pl.BlockSpec((F, F), lambda i: (0, 0))],
        out_specs=pl.BlockSpec((NB_NODE, F), lambda i: (i, 0)),
        out_shape=jax.ShapeDtypeStruct((n_nodes, F), jnp.float32),
    )(node_feat, pmat)

    @functools.partial(
        pl.kernel,
        out_type=jax.ShapeDtypeStruct((n_edges, F), jnp.float32),
        mesh=mesh,
        scratch_types=[pltpu.VMEM((K,), jnp.int32),
                       pltpu.VMEM((K, F), jnp.float32),
                       pltpu.SemaphoreType.DMA],
        compiler_params=pltpu.CompilerParams(use_tc_tiling_on_sc=False),
    )
    def _gather(perm_hbm, srcr_hbm, x_hbm, idx_v, rows_v, sem):
        wid = lax.axis_index("s") * NC + lax.axis_index("c")

        def body(q, carry):
            cq = wid + q * NW

            @pl.when(cq < nch)
            def _():
                pltpu.sync_copy(srcr_hbm.at[cq], idx_v)
                pltpu.async_copy(perm_hbm.at[idx_v], rows_v, sem).wait()
                off = pl.multiple_of(cq * K, K)
                pltpu.sync_copy(rows_v, x_hbm.at[pl.ds(off, K)])
            return carry

        lax.fori_loop(0, per_w, body, 0)

    x_src = _gather(perm, src_r)

    msg = pl.pallas_call(
        _msg_body,
        grid=(n_edges // EB_EDGE,),
        in_specs=[pl.BlockSpec((EB_EDGE, 3), lambda i: (i, 0)),
                  pl.BlockSpec((EB_EDGE, F), lambda i: (i, 0)),
                  pl.BlockSpec(W1.shape, lambda i: (0, 0)),
                  pl.BlockSpec(W2.shape, lambda i: (0, 0)),
                  pl.BlockSpec(W3.shape, lambda i: (0, 0))],
        out_specs=pl.BlockSpec((EB_EDGE, F), lambda i: (i, 0)),
        out_shape=jax.ShapeDtypeStruct((n_edges, F), jnp.float32),
    )(edge_vec, x_src, W1, W2, W3)

    zinit = jnp.zeros((n_pad, HALF), jnp.float32)

    @functools.partial(
        pl.kernel,
        out_type=jax.ShapeDtypeStruct((NC, n_pad, HALF), jnp.float32),
        mesh=mesh,
        scratch_types=[pltpu.VMEM((K,), jnp.int32),
                       pltpu.VMEM((K, HALF), jnp.float32),
                       pltpu.VMEM_SHARED((n_pad, HALF), jnp.float32)],
        compiler_params=pltpu.CompilerParams(use_tc_tiling_on_sc=False),
    )
    def _scatter(dstr_hbm, msg_hbm, z_hbm, out_hbm, didx_v, mbuf_v, acc_sh):
        cid = lax.axis_index("c")
        sid = lax.axis_index("s")
        row0 = sid * rows_t
        pltpu.sync_copy(z_hbm.at[pl.ds(row0, rows_t)],
                        acc_sh.at[pl.ds(row0, rows_t)])
        plsc.subcore_barrier()

        def body(q, carry):
            cq = q * NS + sid

            @pl.when(cq < nch)
            def _():
                pltpu.sync_copy(dstr_hbm.at[cq], didx_v)
                off = pl.multiple_of(cq * K, K)
                pltpu.sync_copy(
                    msg_hbm.at[pl.ds(off, K), pl.ds(cid * HALF, HALF)], mbuf_v)
                pltpu.sync_copy(mbuf_v, acc_sh.at[didx_v], add=True)
            return carry

        lax.fori_loop(0, per_t, body, 0)
        plsc.subcore_barrier()
        pltpu.sync_copy(acc_sh.at[pl.ds(row0, rows_t)],
                        out_hbm.at[cid, pl.ds(row0, rows_t)])

    aggr2 = _scatter(dst_r, msg, zinit)
    aggr2 = aggr2[:, :n_nodes, :]

    wbig = _folded_linear(Wl0, Wl1)
    out = pl.pallas_call(
        _final_body,
        grid=(n_nodes // NB_NODE,),
        in_specs=[pl.BlockSpec((NC, NB_NODE, HALF), lambda i: (0, i, 0)),
                  pl.BlockSpec((F, F), lambda i: (0, 0)),
                  pl.BlockSpec((NB_NODE, F), lambda i: (i, 0))],
        out_specs=pl.BlockSpec((NB_NODE, F), lambda i: (i, 0)),
        out_shape=jax.ShapeDtypeStruct((n_nodes, F), jnp.float32),
    )(aggr2, wbig, node_feat)
    return out


# speedup vs baseline: 3.2734x; 2.3342x over previous
import functools

import numpy as np
import jax
import jax.numpy as jnp
from jax import lax
from jax.experimental import pallas as pl
from jax.experimental.pallas import tpu as pltpu
from jax.experimental.pallas import tpu_sc as plsc

MUL = 16
F = 4 * MUL
SQRT2 = 2.0 ** 0.5
SQRT3 = 3.0 ** 0.5
SQRT5 = 5.0 ** 0.5

NC, NS = 2, 16
NW = NC * NS
K = 128
HALF = F // 2

NB_NODE = 2000
EB_EDGE = 3200


def _permute_body(x_ref, p_ref, o_ref):
    o_ref[...] = jnp.dot(x_ref[...], p_ref[...],
                         preferred_element_type=jnp.float32)


def _final_body(a_ref, wb_ref, nf_ref, o_ref):
    wb = wb_ref[...]
    lin = jnp.dot(a_ref[0], wb[:HALF, :], preferred_element_type=jnp.float32)
    lin += jnp.dot(a_ref[1], wb[HALF:, :], preferred_element_type=jnp.float32)
    o_ref[...] = lin + nf_ref[...]


def _tdot(lhs_t, rhs):
    return lax.dot_general(lhs_t, rhs, (((0,), (0,)), ((), ())),
                           preferred_element_type=jnp.float32)


def _msg_body(evt_ref, x_ref, w1_ref, w2_ref, w3_ref, q1_ref, q2_ref,
              q3_ref, q4_ref, s0_ref, r_ref, mr1_ref, mr2_ref, esh_ref,
              kd_ref, k1_ref, k2_ref, cdc_ref, o_ref):
    evt = evt_ref[...]
    sq = evt * evt
    r2 = sq[0:1, :] + sq[1:2, :] + sq[2:3, :] + 1e-12
    inv = lax.rsqrt(r2)
    r_t = r2 * inv
    inv2 = inv * inv
    n_t = evt * inv
    na_t = sq * inv2
    nrot_t = jnp.concatenate([n_t[1:3, :], n_t[0:1, :]], axis=0)
    nprod_t = n_t * nrot_t
    h = _tdot(r_t, w1_ref[...])
    h = h * jax.nn.sigmoid(h)
    h = jnp.dot(h, w2_ref[...], preferred_element_type=jnp.float32)
    h = h * jax.nn.sigmoid(h)
    w = jnp.dot(h, w3_ref[...], preferred_element_type=jnp.float32)
    x = x_ref[...]
    cdc = cdc_ref[...]
    she0 = _tdot(n_t, esh_ref[...])
    xs = jnp.dot(x, s0_ref[...], preferred_element_type=jnp.float32)
    d = x * she0
    dr = jnp.dot(d, r_ref[...], preferred_element_type=jnp.float32)
    xr = jnp.dot(x, mr1_ref[...], preferred_element_type=jnp.float32)
    xr2 = jnp.dot(x, mr2_ref[...], preferred_element_type=jnp.float32)
    gd = _tdot(na_t, kd_ref[...]) + cdc[0:1, :]
    g1 = _tdot(nprod_t, k1_ref[...])
    g2 = _tdot(nprod_t, k2_ref[...])
    wq1 = jnp.dot(w, q1_ref[...], preferred_element_type=jnp.float32)
    wq2 = jnp.dot(w, q2_ref[...], preferred_element_type=jnp.float32)
    wq3 = jnp.dot(w, q3_ref[...], preferred_element_type=jnp.float32)
    wq4 = jnp.dot(w, q4_ref[...], preferred_element_type=jnp.float32)
    tv = gd * x + g1 * xr + g2 * xr2
    o_ref[...] = (wq1 * (she0 + cdc[1:2, :]) * xs + wq2 * dr + wq3 * x
                  + wq4 * tv)


SQRT15 = 15.0 ** 0.5


def _msg_consts():
    u = np.arange(MUL)

    def blk(b):
        return MUL * b + u

    q1 = np.zeros((80, F), np.float32)
    q1[u, blk(0)] = 1.0 / SQRT2
    q2 = np.zeros((80, F), np.float32)
    q2[48 + u, blk(0)] = 1.0 / (SQRT3 * SQRT2)
    q3 = np.zeros((80, F), np.float32)
    q4 = np.zeros((80, F), np.float32)
    for c in (1, 2, 3):
        q1[16 + u, blk(c)] = 1.0 / SQRT3
        q3[32 + u, blk(c)] = 1.0 / SQRT3
        q4[64 + u, blk(c)] = 1.0 / (SQRT3 * SQRT5)
    s0 = np.zeros((F, F), np.float32)
    for b in range(4):
        s0[u, blk(b)] = 1.0
    rmat = np.zeros((F, F), np.float32)
    for b in (1, 2, 3):
        rmat[blk(b), u] = 1.0
    mr1 = np.zeros((F, F), np.float32)
    mr1[blk(2), blk(1)] = 1.0
    mr1[blk(3), blk(2)] = 1.0
    mr1[blk(1), blk(3)] = 1.0
    mr2 = np.zeros((F, F), np.float32)
    mr2[blk(3), blk(1)] = 1.0
    mr2[blk(1), blk(2)] = 1.0
    mr2[blk(2), blk(3)] = 1.0
    esh = np.zeros((3, F), np.float32)
    for c in (1, 2, 3):
        esh[c - 1, blk(c)] = SQRT3
    kd = np.zeros((3, F), np.float32)
    kd[0, blk(1)] = SQRT15 / 2
    kd[1, blk(1)] = -SQRT15 / 2
    kd[2, blk(1)] = -SQRT15 / 2
    kd[0, blk(2)] = -SQRT15 / 2
    kd[1, blk(2)] = SQRT15 / 2
    kd[2, blk(2)] = -SQRT15 / 2
    kd[2, blk(3)] = SQRT15
    k1 = np.zeros((3, F), np.float32)
    k1[0, blk(1)] = SQRT15
    k1[1, blk(2)] = SQRT15
    k1[2, blk(3)] = SQRT15
    k2 = np.zeros((3, F), np.float32)
    k2[2, blk(1)] = SQRT15
    k2[0, blk(2)] = SQRT15
    k2[1, blk(3)] = SQRT15
    cdc = np.zeros((2, F), np.float32)
    cdc[0, blk(1)] = SQRT15 / 6
    cdc[0, blk(2)] = SQRT15 / 6
    cdc[0, blk(3)] = -SQRT15 / 3
    cdc[1, blk(0)] = 1.0
    return [q1, q2, q3, q4, s0, rmat, mr1, mr2, esh, kd, k1, k2, cdc]


def _run_msg(edge_vec, x_src, W1, W2, W3, interpret=False):
    n_edges = x_src.shape[0]
    consts = [jnp.asarray(c) for c in _msg_consts()]
    cspecs = [pl.BlockSpec(c.shape, lambda i: (0, 0)) for c in consts]
    return pl.pallas_call(
        _msg_body,
        grid=(n_edges // EB_EDGE,),
        in_specs=[pl.BlockSpec((3, EB_EDGE), lambda i: (0, i)),
                  pl.BlockSpec((EB_EDGE, F), lambda i: (i, 0)),
                  pl.BlockSpec(W1.shape, lambda i: (0, 0)),
                  pl.BlockSpec(W2.shape, lambda i: (0, 0)),
                  pl.BlockSpec(W3.shape, lambda i: (0, 0))] + cspecs,
        out_specs=pl.BlockSpec((EB_EDGE, F), lambda i: (i, 0)),
        out_shape=jax.ShapeDtypeStruct((n_edges, F), jnp.float32),
        interpret=interpret,
    )(edge_vec.T, x_src, W1, W2 * 0.125, W3 * 0.125, *consts)


def _perm_matrix():
    p = np.zeros((F, F), np.float32)
    p[np.arange(MUL), np.arange(MUL)] = 1.0
    for c in range(3):
        for u in range(MUL):
            p[MUL + 3 * u + c, MUL + MUL * c + u] = 1.0
    return p


def _folded_linear(wl0, wl1):
    wb = jnp.zeros((F, F), jnp.float32)
    wb = wb.at[:MUL, :MUL].set(wl0 * 0.25)
    cc, uu, vv = np.meshgrid(np.arange(3), np.arange(MUL), np.arange(MUL),
                             indexing="ij")
    rows = MUL + MUL * cc + uu
    cols = MUL + 3 * vv + cc
    vals = jnp.broadcast_to(wl1 * 0.25, (3, MUL, MUL))
    return wb.at[rows, cols].set(vals)


def kernel(node_feat, edge_index, edge_vec, W1, W2, W3, Wl0, Wl1):
    n_nodes = node_feat.shape[0]
    n_edges = edge_vec.shape[0]
    assert n_edges % K == 0
    nch = n_edges // K
    per_w = -(-nch // NW)
    per_t = -(-nch // NS)
    n_pad = -(-n_nodes // (NS * 8)) * NS * 8
    rows_t = n_pad // NS

    src_r = edge_index[0].reshape(nch, K)
    dst_r = edge_index[1].reshape(nch, K)

    mesh = plsc.VectorSubcoreMesh(core_axis_name="c", subcore_axis_name="s",
                                  num_cores=NC, num_subcores=NS)

    pmat = jnp.asarray(_perm_matrix())
    perm = pl.pallas_call(
        _permute_body,
        grid=(n_nodes // NB_NODE,),
        in_specs=[pl.BlockSpec((NB_NODE, F), lambda i: (i, 0)),
                  pl.BlockSpec((F, F), lambda i: (0, 0))],
        out_specs=pl.BlockSpec((NB_NODE, F), lambda i: (i, 0)),
        out_shape=jax.ShapeDtypeStruct((n_nodes, F), jnp.float32),
    )(node_feat, pmat)

    @functools.partial(
        pl.kernel,
        out_type=jax.ShapeDtypeStruct((n_edges, F), jnp.float32),
        mesh=mesh,
        scratch_types=[pltpu.VMEM((K,), jnp.int32),
                       pltpu.VMEM((K, F), jnp.float32),
                       pltpu.SemaphoreType.DMA],
        compiler_params=pltpu.CompilerParams(use_tc_tiling_on_sc=False),
    )
    def _gather(perm_hbm, srcr_hbm, x_hbm, idx_v, rows_v, sem):
        wid = lax.axis_index("s") * NC + lax.axis_index("c")

        def body(q, carry):
            cq = wid + q * NW

            @pl.when(cq < nch)
            def _():
                pltpu.sync_copy(srcr_hbm.at[cq], idx_v)
                pltpu.async_copy(perm_hbm.at[idx_v], rows_v, sem).wait()
                off = pl.multiple_of(cq * K, K)
                pltpu.sync_copy(rows_v, x_hbm.at[pl.ds(off, K)])
            return carry

        lax.fori_loop(0, per_w, body, 0)

    x_src = _gather(perm, src_r)

    msg = _run_msg(edge_vec, x_src, W1, W2, W3)

    zinit = jnp.zeros((n_pad, HALF), jnp.float32)

    @functools.partial(
        pl.kernel,
        out_type=jax.ShapeDtypeStruct((NC, n_pad, HALF), jnp.float32),
        mesh=mesh,
        scratch_types=[pltpu.VMEM((K,), jnp.int32),
                       pltpu.VMEM((K, HALF), jnp.float32),
                       pltpu.VMEM_SHARED((n_pad, HALF), jnp.float32)],
        compiler_params=pltpu.CompilerParams(use_tc_tiling_on_sc=False),
    )
    def _scatter(dstr_hbm, msg_hbm, z_hbm, out_hbm, didx_v, mbuf_v, acc_sh):
        cid = lax.axis_index("c")
        sid = lax.axis_index("s")
        row0 = sid * rows_t
        pltpu.sync_copy(z_hbm.at[pl.ds(row0, rows_t)],
                        acc_sh.at[pl.ds(row0, rows_t)])
        plsc.subcore_barrier()

        def body(q, carry):
            cq = q * NS + sid

            @pl.when(cq < nch)
            def _():
                pltpu.sync_copy(dstr_hbm.at[cq], didx_v)
                off = pl.multiple_of(cq * K, K)
                pltpu.sync_copy(
                    msg_hbm.at[pl.ds(off, K), pl.ds(cid * HALF, HALF)], mbuf_v)
                pltpu.sync_copy(mbuf_v, acc_sh.at[didx_v], add=True)
            return carry

        lax.fori_loop(0, per_t, body, 0)
        plsc.subcore_barrier()
        pltpu.sync_copy(acc_sh.at[pl.ds(row0, rows_t)],
                        out_hbm.at[cid, pl.ds(row0, rows_t)])

    aggr2 = _scatter(dst_r, msg, zinit)
    aggr2 = aggr2[:, :n_nodes, :]

    wbig = _folded_linear(Wl0, Wl1)
    out = pl.pallas_call(
        _final_body,
        grid=(n_nodes // NB_NODE,),
        in_specs=[pl.BlockSpec((NC, NB_NODE, HALF), lambda i: (0, i, 0)),
                  pl.BlockSpec((F, F), lambda i: (0, 0)),
                  pl.BlockSpec((NB_NODE, F), lambda i: (i, 0))],
        out_specs=pl.BlockSpec((NB_NODE, F), lambda i: (i, 0)),
        out_shape=jax.ShapeDtypeStruct((n_nodes, F), jnp.float32),
    )(aggr2, wbig, node_feat)
    return out


# speedup vs baseline: 4.2954x; 1.3122x over previous
import functools

import numpy as np
import jax
import jax.numpy as jnp
from jax import lax
from jax.experimental import pallas as pl
from jax.experimental.pallas import tpu as pltpu
from jax.experimental.pallas import tpu_sc as plsc

MUL = 16
F = 4 * MUL
SQRT2 = 2.0 ** 0.5
SQRT3 = 3.0 ** 0.5
SQRT5 = 5.0 ** 0.5

NC, NS = 2, 16
NW = NC * NS
K = 128
G = 5
GK = G * K
GS = 2
GKS = GS * K
HALF = F // 2

NB_NODE = 2000
EB_EDGE = 3200


def _permute_body(x_ref, p_ref, o_ref):
    o_ref[...] = jnp.dot(x_ref[...], p_ref[...],
                         preferred_element_type=jnp.float32)


def _final_body(a_ref, wb_ref, nf_ref, o_ref):
    wb = wb_ref[...]
    lin = jnp.dot(a_ref[0], wb[:HALF, :], preferred_element_type=jnp.float32)
    lin += jnp.dot(a_ref[1], wb[HALF:, :], preferred_element_type=jnp.float32)
    o_ref[...] = lin + nf_ref[...]


def _tdot(lhs_t, rhs):
    return lax.dot_general(lhs_t, rhs, (((0,), (0,)), ((), ())),
                           preferred_element_type=jnp.float32)


def _msg_body(evt_ref, x_ref, w1_ref, w2_ref, w3_ref, q1_ref, q2_ref,
              q3_ref, q4_ref, s0_ref, r_ref, mr1_ref, mr2_ref, esh_ref,
              kd_ref, k1_ref, k2_ref, cdc_ref, pm_ref, o_ref):
    evt = evt_ref[...]
    sq = evt * evt
    r2 = sq[0:1, :] + sq[1:2, :] + sq[2:3, :] + 1e-12
    inv = lax.rsqrt(r2)
    r_t = r2 * inv
    inv2 = inv * inv
    n_t = evt * inv
    na_t = sq * inv2
    nrot_t = jnp.concatenate([n_t[1:3, :], n_t[0:1, :]], axis=0)
    nprod_t = n_t * nrot_t
    h = _tdot(r_t, w1_ref[...])
    h = h * jax.nn.sigmoid(h)
    h = jnp.dot(h, w2_ref[...], preferred_element_type=jnp.float32)
    h = h * jax.nn.sigmoid(h)
    w = jnp.dot(h, w3_ref[...], preferred_element_type=jnp.float32)
    x = jnp.dot(x_ref[...], pm_ref[...], preferred_element_type=jnp.float32)
    cdc = cdc_ref[...]
    she0 = _tdot(n_t, esh_ref[...])
    xs = jnp.dot(x, s0_ref[...], preferred_element_type=jnp.float32)
    d = x * she0
    dr = jnp.dot(d, r_ref[...], preferred_element_type=jnp.float32)
    xr = jnp.dot(x, mr1_ref[...], preferred_element_type=jnp.float32)
    xr2 = jnp.dot(x, mr2_ref[...], preferred_element_type=jnp.float32)
    gd = _tdot(na_t, kd_ref[...]) + cdc[0:1, :]
    g1 = _tdot(nprod_t, k1_ref[...])
    g2 = _tdot(nprod_t, k2_ref[...])
    wq1 = jnp.dot(w, q1_ref[...], preferred_element_type=jnp.float32)
    wq2 = jnp.dot(w, q2_ref[...], preferred_element_type=jnp.float32)
    wq3 = jnp.dot(w, q3_ref[...], preferred_element_type=jnp.float32)
    wq4 = jnp.dot(w, q4_ref[...], preferred_element_type=jnp.float32)
    tv = gd * x + g1 * xr + g2 * xr2
    o_ref[...] = (wq1 * (she0 + cdc[1:2, :]) * xs + wq2 * dr + wq3 * x
                  + wq4 * tv)


SQRT15 = 15.0 ** 0.5


def _msg_consts():
    u = np.arange(MUL)

    def blk(b):
        return MUL * b + u

    q1 = np.zeros((80, F), np.float32)
    q1[u, blk(0)] = 1.0 / SQRT2
    q2 = np.zeros((80, F), np.float32)
    q2[48 + u, blk(0)] = 1.0 / (SQRT3 * SQRT2)
    q3 = np.zeros((80, F), np.float32)
    q4 = np.zeros((80, F), np.float32)
    for c in (1, 2, 3):
        q1[16 + u, blk(c)] = 1.0 / SQRT3
        q3[32 + u, blk(c)] = 1.0 / SQRT3
        q4[64 + u, blk(c)] = 1.0 / (SQRT3 * SQRT5)
    s0 = np.zeros((F, F), np.float32)
    for b in range(4):
        s0[u, blk(b)] = 1.0
    rmat = np.zeros((F, F), np.float32)
    for b in (1, 2, 3):
        rmat[blk(b), u] = 1.0
    mr1 = np.zeros((F, F), np.float32)
    mr1[blk(2), blk(1)] = 1.0
    mr1[blk(3), blk(2)] = 1.0
    mr1[blk(1), blk(3)] = 1.0
    mr2 = np.zeros((F, F), np.float32)
    mr2[blk(3), blk(1)] = 1.0
    mr2[blk(1), blk(2)] = 1.0
    mr2[blk(2), blk(3)] = 1.0
    esh = np.zeros((3, F), np.float32)
    for c in (1, 2, 3):
        esh[c - 1, blk(c)] = SQRT3
    kd = np.zeros((3, F), np.float32)
    kd[0, blk(1)] = SQRT15 / 2
    kd[1, blk(1)] = -SQRT15 / 2
    kd[2, blk(1)] = -SQRT15 / 2
    kd[0, blk(2)] = -SQRT15 / 2
    kd[1, blk(2)] = SQRT15 / 2
    kd[2, blk(2)] = -SQRT15 / 2
    kd[2, blk(3)] = SQRT15
    k1 = np.zeros((3, F), np.float32)
    k1[0, blk(1)] = SQRT15
    k1[1, blk(2)] = SQRT15
    k1[2, blk(3)] = SQRT15
    k2 = np.zeros((3, F), np.float32)
    k2[2, blk(1)] = SQRT15
    k2[0, blk(2)] = SQRT15
    k2[1, blk(3)] = SQRT15
    cdc = np.zeros((2, F), np.float32)
    cdc[0, blk(1)] = SQRT15 / 6
    cdc[0, blk(2)] = SQRT15 / 6
    cdc[0, blk(3)] = -SQRT15 / 3
    cdc[1, blk(0)] = 1.0
    return [q1, q2, q3, q4, s0, rmat, mr1, mr2, esh, kd, k1, k2, cdc]


def _run_msg(edge_vec, x_src, W1, W2, W3, interpret=False):
    n_edges = x_src.shape[0]
    consts = [jnp.asarray(c) for c in _msg_consts()]
    consts.append(jnp.asarray(_perm_matrix()))
    cspecs = [pl.BlockSpec(c.shape, lambda i: (0, 0)) for c in consts]
    return pl.pallas_call(
        _msg_body,
        grid=(n_edges // EB_EDGE,),
        in_specs=[pl.BlockSpec((3, EB_EDGE), lambda i: (0, i)),
                  pl.BlockSpec((EB_EDGE, F), lambda i: (i, 0)),
                  pl.BlockSpec(W1.shape, lambda i: (0, 0)),
                  pl.BlockSpec(W2.shape, lambda i: (0, 0)),
                  pl.BlockSpec(W3.shape, lambda i: (0, 0))] + cspecs,
        out_specs=pl.BlockSpec((EB_EDGE, F), lambda i: (i, 0)),
        out_shape=jax.ShapeDtypeStruct((n_edges, F), jnp.float32),
        interpret=interpret,
    )(edge_vec.T, x_src, W1, W2 * 0.125, W3 * 0.125, *consts)


def _perm_matrix():
    p = np.zeros((F, F), np.float32)
    p[np.arange(MUL), np.arange(MUL)] = 1.0
    for c in range(3):
        for u in range(MUL):
            p[MUL + 3 * u + c, MUL + MUL * c + u] = 1.0
    return p


def _folded_linear(wl0, wl1):
    wb = jnp.zeros((F, F), jnp.float32)
    wb = wb.at[:MUL, :MUL].set(wl0 * 0.25)
    cc, uu, vv = np.meshgrid(np.arange(3), np.arange(MUL), np.arange(MUL),
                             indexing="ij")
    rows = MUL + MUL * cc + uu
    cols = MUL + 3 * vv + cc
    vals = jnp.broadcast_to(wl1 * 0.25, (3, MUL, MUL))
    return wb.at[rows, cols].set(vals)


def kernel(node_feat, edge_index, edge_vec, W1, W2, W3, Wl0, Wl1):
    n_nodes = node_feat.shape[0]
    n_edges = edge_vec.shape[0]
    assert n_edges % (K * G) == 0 and n_edges % (K * GS) == 0
    nch = n_edges // K
    ng = nch // G
    ngs = nch // GS
    pw_g = (-(-ng // NW) + 1) // 2 * 2
    pt_g = (-(-ngs // NS) + 1) // 2 * 2
    n_pad = -(-n_nodes // (NS * 8)) * NS * 8
    rows_t = n_pad // NS

    src_r = edge_index[0].reshape(nch, K)
    dst_r = edge_index[1].reshape(nch, K)

    mesh = plsc.VectorSubcoreMesh(core_axis_name="c", subcore_axis_name="s",
                                  num_cores=NC, num_subcores=NS)
    scp = pltpu.CompilerParams(use_tc_tiling_on_sc=False)

    @functools.partial(
        pl.kernel,
        out_type=jax.ShapeDtypeStruct((n_edges, F), jnp.float32),
        mesh=mesh,
        scratch_types=[pltpu.VMEM((2, G, K), jnp.int32),
                       pltpu.VMEM((2, GK, F), jnp.float32),
                       pltpu.SemaphoreType.DMA((2,)),
                       pltpu.SemaphoreType.DMA((2,)),
                       pltpu.SemaphoreType.DMA((2,))],
        compiler_params=scp,
    )
    def _gather(nf_hbm, srcr_hbm, x_hbm, idxb, rowsb, isem, gsem, wsem):
        wid = lax.axis_index("s") * NC + lax.axis_index("c")

        def idx_copy(g, b):
            return pltpu.make_async_copy(srcr_hbm.at[pl.ds(g * G, G)],
                                         idxb.at[b], isem.at[b])

        def row_write(g, b):
            return pltpu.make_async_copy(rowsb.at[b],
                                         x_hbm.at[pl.ds(g * GK, GK)],
                                         wsem.at[b])

        def gather_drain(b):
            return pltpu.make_async_copy(nf_hbm.at[pl.ds(0, GK)],
                                         rowsb.at[b], gsem.at[b])

        @pl.when(wid < ng)
        def _():
            idx_copy(wid, 0).start()

        def body(q, carry):
            for b in (0, 1):
                gi = q * 2 + b
                g = wid + gi * NW

                @pl.when(g + NW < ng)
                def _():
                    idx_copy(g + NW, 1 - b).start()

                @pl.when((gi >= 2) & (g - 2 * NW < ng))
                def _():
                    row_write(g - 2 * NW, b).wait()

                @pl.when(g < ng)
                def _():
                    idx_copy(g, b).wait()
                    for j in range(G):
                        pltpu.async_copy(nf_hbm.at[idxb.at[b, j]],
                                         rowsb.at[b, pl.ds(j * K, K)],
                                         gsem.at[b])
                    gather_drain(b).wait()
                    row_write(g, b).start()
            return carry

        lax.fori_loop(0, pw_g // 2, body, 0)
        for t in (pw_g - 2, pw_g - 1):
            g = wid + t * NW

            @pl.when(g < ng)
            def _():
                row_write(g, t % 2).wait()

    x_src = _gather(node_feat, src_r)

    msg = _run_msg(edge_vec, x_src, W1, W2, W3)

    zinit = jnp.zeros((n_pad, HALF), jnp.float32)

    @functools.partial(
        pl.kernel,
        out_type=jax.ShapeDtypeStruct((NC, n_pad, HALF), jnp.float32),
        mesh=mesh,
        scratch_types=[pltpu.VMEM((2, GS, K), jnp.int32),
                       pltpu.VMEM((2, GKS, HALF), jnp.float32),
                       pltpu.VMEM_SHARED((n_pad, HALF), jnp.float32),
                       pltpu.SemaphoreType.DMA((2,)),
                       pltpu.SemaphoreType.DMA((2,)),
                       pltpu.SemaphoreType.DMA((2,))],
        compiler_params=scp,
    )
    def _scatter(dstr_hbm, msg_hbm, z_hbm, out_hbm, didxb, mb, acc_sh,
                 isem, msem, ssem):
        cid = lax.axis_index("c")
        sid = lax.axis_index("s")
        row0 = sid * rows_t
        pltpu.sync_copy(z_hbm.at[pl.ds(row0, rows_t)],
                        acc_sh.at[pl.ds(row0, rows_t)])
        plsc.subcore_barrier()

        def idx_copy(g, b):
            return pltpu.make_async_copy(dstr_hbm.at[pl.ds(g * GS, GS)],
                                         didxb.at[b], isem.at[b])

        def msg_copy(g, b):
            return pltpu.make_async_copy(
                msg_hbm.at[pl.ds(g * GKS, GKS), pl.ds(cid * HALF, HALF)],
                mb.at[b], msem.at[b])

        def scat_drain(b):
            return pltpu.make_async_copy(mb.at[b], acc_sh.at[pl.ds(0, GKS)],
                                         ssem.at[b])

        @pl.when(sid < ngs)
        def _():
            idx_copy(sid, 0).start()
            msg_copy(sid, 0).start()

        def body(q, carry):
            for b in (0, 1):
                gi = q * 2 + b
                g = sid + gi * NS

                @pl.when((gi >= 1) & (g - NS < ngs))
                def _():
                    scat_drain(1 - b).wait()

                @pl.when(g + NS < ngs)
                def _():
                    idx_copy(g + NS, 1 - b).start()
                    msg_copy(g + NS, 1 - b).start()

                @pl.when(g < ngs)
                def _():
                    idx_copy(g, b).wait()
                    msg_copy(g, b).wait()
                    for j in range(GS):
                        pltpu.async_copy(mb.at[b, pl.ds(j * K, K)],
                                         acc_sh.at[didxb.at[b, j]],
                                         ssem.at[b], add=True)
            return carry

        lax.fori_loop(0, pt_g // 2, body, 0)
        t = pt_g - 1
        g_last = sid + t * NS

        @pl.when(g_last < ngs)
        def _():
            scat_drain(t % 2).wait()
        plsc.subcore_barrier()
        pltpu.sync_copy(acc_sh.at[pl.ds(row0, rows_t)],
                        out_hbm.at[cid, pl.ds(row0, rows_t)])

    aggr2 = _scatter(dst_r, msg, zinit)

    wbig = _folded_linear(Wl0, Wl1)
    out = pl.pallas_call(
        _final_body,
        grid=(n_nodes // NB_NODE,),
        in_specs=[pl.BlockSpec((NC, NB_NODE, HALF), lambda i: (0, i, 0)),
                  pl.BlockSpec((F, F), lambda i: (0, 0)),
                  pl.BlockSpec((NB_NODE, F), lambda i: (i, 0))],
        out_specs=pl.BlockSpec((NB_NODE, F), lambda i: (i, 0)),
        out_shape=jax.ShapeDtypeStruct((n_nodes, F), jnp.float32),
    )(aggr2, wbig, node_feat)
    return out
